# Initial kernel scaffold; baseline (speedup 1.0000x reference)
import jax
import jax.numpy as jnp
from jax.experimental import pallas as pl


def kernel(txt_embeds, txt_masks, split_traj_embeds, split_traj_vp_lens, traj_vpids, traj_cand_vpids, gmap_vpids, gmap_step_ids, gmap_pos_fts, gmap_lens, W_pos, b_pos, ln_gamma, ln_beta, step_table):
    raise NotImplementedError("write your pallas kernel here")


# speedup vs baseline: 4.7402x; 4.7402x over previous
import functools

import jax
import jax.numpy as jnp
from jax.experimental import pallas as pl
from jax.experimental.pallas import tpu as pltpu

B, T, V, D = 16, 50, 64, 768
M = 64
POSF = 7
MAX_STEPS = 100
EPS = 1e-12
TV = T * V


def _encoder_kernel(x_ref, lens_ref, vpids_ref, mask_ref, cand_ref, sid_ref,
                    pos_ref, wpos_ref, bpos_ref, gam_ref, bet_ref, table_ref,
                    out_ref):
    f32 = jnp.float32
    x = x_ref[0]
    lens = jnp.maximum(lens_ref[0], 1)
    lensf = lens.astype(f32)
    mask_col = mask_ref[0]

    cand = cand_ref[0]
    m_iota = jax.lax.broadcasted_iota(jnp.int32, (TV, M), 1)
    onehot = jnp.where(cand == m_iota, mask_col, 0.0)
    cand_sum = jax.lax.dot_general(
        onehot, x, (((0,), (0,)), ((), ())),
        preferred_element_type=f32)
    cnt = jnp.sum(onehot, axis=0, keepdims=True)
    unvisited = cand_sum / jnp.maximum(cnt.reshape(M, 1), 1.0)

    masked = x * mask_col
    step_sum = jnp.sum(masked.reshape(T, V, D), axis=1)
    step_fts = step_sum / lensf

    vp = vpids_ref[0]
    m_iota_t = jax.lax.broadcasted_iota(jnp.int32, (T, M), 1)
    t_iota = jax.lax.broadcasted_iota(jnp.int32, (T, M), 0)
    hit = (vp + 1) == m_iota_t
    tstar = jnp.max(jnp.where(hit, t_iota + 1, 0), axis=0, keepdims=True)
    tstar_col = tstar.reshape(M, 1)
    vis_mask = tstar_col > 0
    t_iota_m = jax.lax.broadcasted_iota(jnp.int32, (M, T), 1)
    onehot_vis = ((tstar_col - 1) == t_iota_m).astype(f32)
    visited_fts = jnp.dot(onehot_vis, step_fts, preferred_element_type=f32)

    img = jnp.where(vis_mask, visited_fts, unvisited)
    node_iota = jax.lax.broadcasted_iota(jnp.int32, (M, 1), 0)
    img = jnp.where(node_iota == 0, 0.0, img)

    sid = sid_ref[0]
    s_iota = jax.lax.broadcasted_iota(jnp.int32, (M, MAX_STEPS), 1)
    onehot_step = (sid == s_iota).astype(f32)
    step_emb = jnp.dot(onehot_step, table_ref[...], preferred_element_type=f32)

    h = jnp.dot(pos_ref[0], wpos_ref[...], preferred_element_type=f32) + bpos_ref[...]
    mu = jnp.mean(h, axis=1, keepdims=True)
    var = jnp.mean((h - mu) ** 2, axis=1, keepdims=True)
    ln = (h - mu) / jnp.sqrt(var + EPS) * gam_ref[...] + bet_ref[...]

    out_ref[0] = img + step_emb + ln


@jax.jit
def _encode(split_traj_embeds, split_traj_vp_lens, traj_vpids, traj_cand_vpids,
            gmap_step_ids, gmap_pos_fts, W_pos, b_pos, ln_gamma, ln_beta,
            step_table):
    x = split_traj_embeds.reshape(B, TV, D)
    lens = split_traj_vp_lens.reshape(B, T, 1)
    vpids = traj_vpids.reshape(B, T, 1)
    lens_c = jnp.maximum(split_traj_vp_lens, 1)
    mask_flat = (jnp.arange(V)[None, None, :] < lens_c[:, :, None]).astype(
        jnp.float32).reshape(B, TV, 1)
    cand_flat = traj_cand_vpids.reshape(B, TV, 1)
    sid = gmap_step_ids.reshape(B, M, 1)
    pos = jnp.pad(gmap_pos_fts, ((0, 0), (0, 0), (0, 8 - POSF)))
    wpos = jnp.pad(W_pos, ((0, 8 - POSF), (0, 0)))
    bpos = b_pos.reshape(1, D)
    gam = ln_gamma.reshape(1, D)
    bet = ln_beta.reshape(1, D)

    grid = (B,)
    out = pl.pallas_call(
        _encoder_kernel,
        grid=grid,
        in_specs=[
            pl.BlockSpec((1, TV, D), lambda b: (b, 0, 0)),
            pl.BlockSpec((1, T, 1), lambda b: (b, 0, 0)),
            pl.BlockSpec((1, T, 1), lambda b: (b, 0, 0)),
            pl.BlockSpec((1, TV, 1), lambda b: (b, 0, 0)),
            pl.BlockSpec((1, TV, 1), lambda b: (b, 0, 0)),
            pl.BlockSpec((1, M, 1), lambda b: (b, 0, 0)),
            pl.BlockSpec((1, M, 8), lambda b: (b, 0, 0)),
            pl.BlockSpec((8, D), lambda b: (0, 0)),
            pl.BlockSpec((1, D), lambda b: (0, 0)),
            pl.BlockSpec((1, D), lambda b: (0, 0)),
            pl.BlockSpec((1, D), lambda b: (0, 0)),
            pl.BlockSpec((MAX_STEPS, D), lambda b: (0, 0)),
        ],
        out_specs=pl.BlockSpec((1, M, D), lambda b: (b, 0, 0)),
        out_shape=jax.ShapeDtypeStruct((B, M, D), jnp.float32),
    )(x, lens, vpids, mask_flat, cand_flat, sid, pos, wpos, bpos, gam, bet,
      step_table)
    return out


def kernel(txt_embeds, txt_masks, split_traj_embeds, split_traj_vp_lens,
           traj_vpids, traj_cand_vpids, gmap_vpids, gmap_step_ids,
           gmap_pos_fts, gmap_lens, W_pos, b_pos, ln_gamma, ln_beta,
           step_table):
    return _encode(split_traj_embeds, split_traj_vp_lens, traj_vpids,
                   traj_cand_vpids, gmap_step_ids, gmap_pos_fts, W_pos, b_pos,
                   ln_gamma, ln_beta, step_table)
